# baseline (speedup 1.0000x reference)
import functools

import jax
import jax.numpy as jnp
from jax import lax
from jax.experimental import pallas as pl
from jax.experimental.pallas import tpu as pltpu
from jax.experimental.pallas import tpu_sc as plsc

V = 100000
D = 64
B = 1024
C = 20

NC = 2
NS = 16
NW = NC * NS
EPW = B // NW
CPW = EPW * C
ICH = 128
NCH = CPW // ICH

TV = 2048
NVT = (V + TV - 1) // TV
VP = NVT * TV
BT = VP // 128

_SC_PARAMS = pltpu.CompilerParams(use_tc_tiling_on_sc=False)
NEG = -1e30


def _prepw_body(w_ref, o_ref):
  i = pl.program_id(0)
  nv = pl.num_programs(0)
  w = w_ref[...]

  def _mask(wv):
    row = i * TV + lax.broadcasted_iota(jnp.int32, (TV, 1), 0)
    return jnp.where(row < V, wv, 0.0)

  w = lax.cond(i == nv - 1, _mask, lambda wv: wv, w)
  o_ref[...] = w.astype(jnp.bfloat16)


def _prep_w(out_tab):
  return pl.pallas_call(
      _prepw_body,
      grid=(NVT,),
      in_specs=[pl.BlockSpec((TV, D), lambda i: (i, 0))],
      out_specs=pl.BlockSpec((TV, D), lambda i: (i, 0)),
      out_shape=jax.ShapeDtypeStruct((VP, D), jnp.bfloat16),
  )(out_tab)


def _sc_ctx_gather(ctx_ids, in_tab):
  mesh = plsc.VectorSubcoreMesh(core_axis_name="c", subcore_axis_name="s")

  @functools.partial(
      pl.kernel,
      out_type=jax.ShapeDtypeStruct((B, 128), jnp.float32),
      mesh=mesh,
      compiler_params=_SC_PARAMS,
      scratch_types=[
          pltpu.VMEM((CPW,), jnp.int32),
          pltpu.VMEM((CPW, D), jnp.float32),
          pltpu.VMEM((EPW, 128), jnp.float32),
          pltpu.SemaphoreType.DMA,
      ],
  )
  def k(ids_h, tab_h, x_h, idxc, rows, xout, sem):
    wid = lax.axis_index("s") * NC + lax.axis_index("c")
    eb = wid * EPW

    pltpu.sync_copy(ids_h.at[pl.ds(wid * CPW, CPW)], idxc)
    hs = [
        pltpu.async_copy(tab_h.at[idxc.at[pl.ds(j * ICH, ICH)]],
                         rows.at[pl.ds(j * ICH, ICH)], sem)
        for j in range(NCH)
    ]
    for h in hs:
      h.wait()

    one16 = jnp.where(lax.iota(jnp.int32, 16) == 0, 1.0, 0.0)
    z16 = jnp.zeros((16,), jnp.float32)

    def esum(e, carry):
      for d in range(D // 16):
        sl = pl.ds(d * 16, 16)
        acc = rows[e * C, sl]
        for c2 in range(1, C):
          acc = acc + rows[e * C + c2, sl]
        xout[e, sl] = acc
      xout[e, pl.ds(D, 16)] = one16
      for d in range(D // 16 + 1, 128 // 16):
        xout[e, pl.ds(d * 16, 16)] = z16
      return carry

    lax.fori_loop(0, EPW, esum, 0)
    pltpu.sync_copy(xout, x_h.at[pl.ds(eb, EPW)])

  return k(ctx_ids, in_tab)


def _sc_label_gather(out_tab, bias128, lab_ids, brow_ids):
  mesh = plsc.VectorSubcoreMesh(core_axis_name="c", subcore_axis_name="s")

  @functools.partial(
      pl.kernel,
      out_type=(
          jax.ShapeDtypeStruct((B, 128), jnp.float32),
          jax.ShapeDtypeStruct((B, 128), jnp.float32),
      ),
      mesh=mesh,
      compiler_params=_SC_PARAMS,
      scratch_types=[
          pltpu.VMEM((EPW,), jnp.int32),
          pltpu.VMEM((EPW, D), jnp.float32),
          pltpu.VMEM((EPW,), jnp.int32),
          pltpu.VMEM((EPW, 128), jnp.float32),
          pltpu.VMEM((EPW, 128), jnp.float32),
          pltpu.SemaphoreType.DMA,
      ],
  )
  def k(tab_h, b128_h, lab_h, rid_h, le_h, br_h,
        idxl, labv, ridv, brv, mix, sem):
    wid = lax.axis_index("s") * NC + lax.axis_index("c")
    eb = wid * EPW
    pltpu.sync_copy(lab_h.at[pl.ds(eb, EPW)], idxl)
    pltpu.sync_copy(rid_h.at[pl.ds(eb, EPW)], ridv)
    h2 = pltpu.async_copy(tab_h.at[idxl], labv, sem)
    h3 = pltpu.async_copy(b128_h.at[ridv], brv, sem)
    h2.wait()
    h3.wait()

    z16 = jnp.zeros((16,), jnp.float32)

    def pack(e, carry):
      for d in range(D // 16):
        mix[e, pl.ds(d * 16, 16)] = labv[e, pl.ds(d * 16, 16)]
      for d in range(D // 16, 128 // 16):
        mix[e, pl.ds(d * 16, 16)] = z16
      return carry

    lax.fori_loop(0, EPW, pack, 0)
    pltpu.sync_copy(mix, le_h.at[pl.ds(eb, EPW)])
    pltpu.sync_copy(brv, br_h.at[pl.ds(eb, EPW)])

  return k(out_tab, bias128, lab_ids, brow_ids)


def _lse_body(x_ref, w_ref, b_ref, o_ref, xb_ref, m_ref, s_ref):
  i = pl.program_id(0)
  nv = pl.num_programs(0)

  @pl.when(i == 0)
  def _():
    xb_ref[...] = x_ref[:, 0:D].astype(jnp.bfloat16)
    m_ref[...] = jnp.full((B, 1), NEG, jnp.float32)
    s_ref[...] = jnp.zeros((B, 1), jnp.float32)

  t = lax.dot_general(xb_ref[...], w_ref[...], (((1,), (1,)), ((), ())),
                      preferred_element_type=jnp.float32).astype(jnp.bfloat16)
  t = t + b_ref[...]
  tmax = jnp.max(t, axis=1, keepdims=True).astype(jnp.float32)
  m_old = m_ref[...]
  m_new = jnp.maximum(m_old, tmax)
  u = t - m_new.astype(jnp.bfloat16)
  p = jnp.sum(jnp.exp(u).astype(jnp.float32), axis=1, keepdims=True)
  s_ref[...] = s_ref[...] * jnp.exp(m_old - m_new) + p
  m_ref[...] = m_new

  @pl.when(i == nv - 1)
  def _():
    o_ref[...] = m_ref[...] + jnp.log(s_ref[...])


def _tc_lse(x_aug, w_bf, bias2):
  return pl.pallas_call(
      _lse_body,
      grid=(NVT,),
      in_specs=[
          pl.BlockSpec((B, 128), lambda i: (0, 0)),
          pl.BlockSpec((TV, D), lambda i: (i, 0)),
          pl.BlockSpec((1, TV), lambda i: (0, i)),
      ],
      out_specs=pl.BlockSpec((B, 1), lambda i: (0, 0)),
      out_shape=jax.ShapeDtypeStruct((B, 1), jnp.float32),
      scratch_shapes=[
          pltpu.VMEM((B, D), jnp.bfloat16),
          pltpu.VMEM((B, 1), jnp.float32),
          pltpu.VMEM((B, 1), jnp.float32),
      ],
  )(x_aug, w_bf, bias2)


def _final_body(lse_ref, x_ref, le_ref, br_ref, bl_ref, o_ref):
  xa = x_ref[:, 0:D]
  lab_logit = jnp.sum(xa * le_ref[:, 0:D], axis=1, keepdims=True)
  lsel = lax.broadcasted_iota(jnp.int32, (B, 128), 1) == bl_ref[...]
  lbias = jnp.sum(jnp.where(lsel, br_ref[...], 0.0), axis=1, keepdims=True)
  o_ref[...] = lse_ref[...] - lab_logit - lbias


def _tc_final(lse, x_aug, labv, brv, blane2):
  return pl.pallas_call(
      _final_body,
      out_shape=jax.ShapeDtypeStruct((B, 1), jnp.float32),
  )(lse, x_aug, labv, brv, blane2)


def kernel(input_word_ids, output_word_ids, input_layer_embeddings,
           output_layer_embeddings, output_layer_bias):
  ctx = input_word_ids.reshape(B * C)
  brow_ids = lax.shift_right_logical(output_word_ids, 7)
  blane = lax.bitwise_and(output_word_ids, 127)
  bias128 = jnp.pad(output_layer_bias, (0, VP - V)).reshape(BT, 128)
  bias2 = jnp.pad(output_layer_bias, (0, VP - V),
                  constant_values=NEG).reshape(1, VP).astype(jnp.bfloat16)

  w_bf = _prep_w(output_layer_embeddings)
  x_aug = _sc_ctx_gather(ctx, input_layer_embeddings)
  labv, brv = _sc_label_gather(output_layer_embeddings, bias128,
                               output_word_ids, brow_ids)

  lse = _tc_lse(x_aug, w_bf, bias2)
  loss = _tc_final(lse, x_aug, labv, brv, blane.reshape(B, 1))
  return loss.reshape(B)


# speedup vs baseline: 1.0297x; 1.0297x over previous
import functools

import jax
import jax.numpy as jnp
from jax import lax
from jax.experimental import pallas as pl
from jax.experimental.pallas import tpu as pltpu
from jax.experimental.pallas import tpu_sc as plsc

V = 100000
D = 64
B = 1024
C = 20

NC = 2
NS = 16
NW = NC * NS
EPW = B // NW
CPW = EPW * C
ICH = 128
NCH = CPW // ICH

VH = V // 2
TP = 1024
NPT = (VH + TP - 1) // TP
HP = NPT * TP
VP = 2 * HP
BT = VP // 128

_SC_PARAMS = pltpu.CompilerParams(use_tc_tiling_on_sc=False)
NEG = -1e30


def _sc_ctx_gather(ctx_ids, in_tab):
  mesh = plsc.VectorSubcoreMesh(core_axis_name="c", subcore_axis_name="s")

  @functools.partial(
      pl.kernel,
      out_type=jax.ShapeDtypeStruct((B, 128), jnp.float32),
      mesh=mesh,
      compiler_params=_SC_PARAMS,
      scratch_types=[
          pltpu.VMEM((CPW,), jnp.int32),
          pltpu.VMEM((CPW, D), jnp.float32),
          pltpu.VMEM((EPW, 128), jnp.float32),
          pltpu.SemaphoreType.DMA,
      ],
  )
  def k(ids_h, tab_h, x_h, idxc, rows, xout, sem):
    wid = lax.axis_index("s") * NC + lax.axis_index("c")
    eb = wid * EPW

    pltpu.sync_copy(ids_h.at[pl.ds(wid * CPW, CPW)], idxc)
    hs = [
        pltpu.async_copy(tab_h.at[idxc.at[pl.ds(j * ICH, ICH)]],
                         rows.at[pl.ds(j * ICH, ICH)], sem)
        for j in range(NCH)
    ]
    for h in hs:
      h.wait()

    one16 = jnp.where(lax.iota(jnp.int32, 16) == 0, 1.0, 0.0)
    z16 = jnp.zeros((16,), jnp.float32)

    def esum(e, carry):
      for d in range(D // 16):
        sl = pl.ds(d * 16, 16)
        acc = rows[e * C, sl]
        for c2 in range(1, C):
          acc = acc + rows[e * C + c2, sl]
        xout[e, sl] = acc
      xout[e, pl.ds(D, 16)] = one16
      for d in range(D // 16 + 1, 128 // 16):
        xout[e, pl.ds(d * 16, 16)] = z16
      return carry

    lax.fori_loop(0, EPW, esum, 0)
    pltpu.sync_copy(xout, x_h.at[pl.ds(eb, EPW)])

  return k(ctx_ids, in_tab)


def _sc_label_gather(out_tab, bias128, lab_ids, brow_ids):
  mesh = plsc.VectorSubcoreMesh(core_axis_name="c", subcore_axis_name="s")

  @functools.partial(
      pl.kernel,
      out_type=(
          jax.ShapeDtypeStruct((B, 128), jnp.float32),
          jax.ShapeDtypeStruct((B, 128), jnp.float32),
      ),
      mesh=mesh,
      compiler_params=_SC_PARAMS,
      scratch_types=[
          pltpu.VMEM((EPW,), jnp.int32),
          pltpu.VMEM((EPW, D), jnp.float32),
          pltpu.VMEM((EPW,), jnp.int32),
          pltpu.VMEM((EPW, 128), jnp.float32),
          pltpu.VMEM((EPW, 128), jnp.float32),
          pltpu.SemaphoreType.DMA,
      ],
  )
  def k(tab_h, b128_h, lab_h, rid_h, le_h, br_h,
        idxl, labv, ridv, brv, mix, sem):
    wid = lax.axis_index("s") * NC + lax.axis_index("c")
    eb = wid * EPW
    pltpu.sync_copy(lab_h.at[pl.ds(eb, EPW)], idxl)
    pltpu.sync_copy(rid_h.at[pl.ds(eb, EPW)], ridv)
    h2 = pltpu.async_copy(tab_h.at[idxl], labv, sem)
    h3 = pltpu.async_copy(b128_h.at[ridv], brv, sem)
    h2.wait()
    h3.wait()

    z16 = jnp.zeros((16,), jnp.float32)

    def pack(e, carry):
      for d in range(D // 16):
        mix[e, pl.ds(d * 16, 16)] = labv[e, pl.ds(d * 16, 16)]
      for d in range(D // 16, 128 // 16):
        mix[e, pl.ds(d * 16, 16)] = z16
      return carry

    lax.fori_loop(0, EPW, pack, 0)
    pltpu.sync_copy(mix, le_h.at[pl.ds(eb, EPW)])
    pltpu.sync_copy(brv, br_h.at[pl.ds(eb, EPW)])

  return k(out_tab, bias128, lab_ids, brow_ids)


def _lse_body(x_ref, w_ref, be_ref, bo_ref, o_ref, xb_ref, m_ref, s_ref):
  i = pl.program_id(0)
  nv = pl.num_programs(0)

  @pl.when(i == 0)
  def _():
    xb_ref[...] = x_ref[:, 0:D].astype(jnp.bfloat16)
    m_ref[...] = jnp.full((B, 1), NEG, jnp.float32)
    s_ref[...] = jnp.zeros((B, 1), jnp.float32)

  xb = xb_ref[...]
  w = w_ref[...]

  def _mask(wv):
    row = i * TP + lax.broadcasted_iota(jnp.int32, (TP, 1), 0)
    return jnp.where(row < VH, wv, jnp.bfloat16(0))

  w = lax.cond(i == nv - 1, _mask, lambda wv: wv, w)
  te = lax.dot_general(xb, w[:, 0:D], (((1,), (1,)), ((), ())),
                       preferred_element_type=jnp.float32).astype(jnp.bfloat16)
  to = lax.dot_general(xb, w[:, D:2 * D], (((1,), (1,)), ((), ())),
                       preferred_element_type=jnp.float32).astype(jnp.bfloat16)
  te = te + be_ref[...]
  to = to + bo_ref[...]
  tmax = jnp.maximum(
      jnp.max(te, axis=1, keepdims=True),
      jnp.max(to, axis=1, keepdims=True)).astype(jnp.float32)
  m_old = m_ref[...]
  m_new = jnp.maximum(m_old, tmax)
  mb = m_new.astype(jnp.bfloat16)
  p = (jnp.sum(jnp.exp(te - mb).astype(jnp.float32), axis=1, keepdims=True) +
       jnp.sum(jnp.exp(to - mb).astype(jnp.float32), axis=1, keepdims=True))
  s_ref[...] = s_ref[...] * jnp.exp(m_old - m_new) + p
  m_ref[...] = m_new

  @pl.when(i == nv - 1)
  def _():
    o_ref[...] = m_ref[...] + jnp.log(s_ref[...])


def _tc_lse(x_aug, w_pairb, be, bo):
  return pl.pallas_call(
      _lse_body,
      grid=(NPT,),
      in_specs=[
          pl.BlockSpec((B, 128), lambda i: (0, 0)),
          pl.BlockSpec((TP, 128), lambda i: (i, 0)),
          pl.BlockSpec((1, TP), lambda i: (0, i)),
          pl.BlockSpec((1, TP), lambda i: (0, i)),
      ],
      out_specs=pl.BlockSpec((B, 1), lambda i: (0, 0)),
      out_shape=jax.ShapeDtypeStruct((B, 1), jnp.float32),
      scratch_shapes=[
          pltpu.VMEM((B, D), jnp.bfloat16),
          pltpu.VMEM((B, 1), jnp.float32),
          pltpu.VMEM((B, 1), jnp.float32),
      ],
  )(x_aug, w_pairb, be, bo)


def _final_body(lse_ref, x_ref, le_ref, br_ref, bl_ref, o_ref):
  xa = x_ref[:, 0:D]
  lab_logit = jnp.sum(xa * le_ref[:, 0:D], axis=1, keepdims=True)
  lsel = lax.broadcasted_iota(jnp.int32, (B, 128), 1) == bl_ref[...]
  lbias = jnp.sum(jnp.where(lsel, br_ref[...], 0.0), axis=1, keepdims=True)
  o_ref[...] = lse_ref[...] - lab_logit - lbias


def _tc_final(lse, x_aug, labv, brv, blane2):
  return pl.pallas_call(
      _final_body,
      out_shape=jax.ShapeDtypeStruct((B, 1), jnp.float32),
  )(lse, x_aug, labv, brv, blane2)


def kernel(input_word_ids, output_word_ids, input_layer_embeddings,
           output_layer_embeddings, output_layer_bias):
  ctx = input_word_ids.reshape(B * C)
  brow_ids = lax.shift_right_logical(output_word_ids, 7)
  blane = lax.bitwise_and(output_word_ids, 127)
  bias128 = jnp.pad(output_layer_bias, (0, VP - V)).reshape(BT, 128)
  be = jnp.pad(output_layer_bias[0::2], (0, HP - VH),
               constant_values=NEG).reshape(1, HP).astype(jnp.bfloat16)
  bo = jnp.pad(output_layer_bias[1::2], (0, HP - VH),
               constant_values=NEG).reshape(1, HP).astype(jnp.bfloat16)

  w_pairb = output_layer_embeddings.astype(jnp.bfloat16).reshape(VH, 128)

  x_aug = _sc_ctx_gather(ctx, input_layer_embeddings)
  labv, brv = _sc_label_gather(output_layer_embeddings, bias128,
                               output_word_ids, brow_ids)

  lse = _tc_lse(x_aug, w_pairb, be, bo)
  loss = _tc_final(lse, x_aug, labv, brv, blane.reshape(B, 1))
  return loss.reshape(B)


# speedup vs baseline: 1.0443x; 1.0142x over previous
import functools

import jax
import jax.numpy as jnp
from jax import lax
from jax.experimental import pallas as pl
from jax.experimental.pallas import tpu as pltpu
from jax.experimental.pallas import tpu_sc as plsc

V = 100000
D = 64
B = 1024
C = 20

NC = 2
NS = 16
NW = NC * NS
EPW = B // NW
CPW = EPW * C
ICH = 128
NCH = CPW // ICH

TV = 2048
NVT = (V + TV - 1) // TV
VP = NVT * TV
BT = VP // 128

_SC_PARAMS = pltpu.CompilerParams(use_tc_tiling_on_sc=False)
NEG = -1e30


def _sc_ctx_gather(ctx_ids, in_tab):
  mesh = plsc.VectorSubcoreMesh(core_axis_name="c", subcore_axis_name="s")

  @functools.partial(
      pl.kernel,
      out_type=jax.ShapeDtypeStruct((B, 128), jnp.float32),
      mesh=mesh,
      compiler_params=_SC_PARAMS,
      scratch_types=[
          pltpu.VMEM((CPW,), jnp.int32),
          pltpu.VMEM((CPW, D), jnp.float32),
          pltpu.VMEM((EPW, 128), jnp.float32),
          pltpu.SemaphoreType.DMA,
      ],
  )
  def k(ids_h, tab_h, x_h, idxc, rows, xout, sem):
    wid = lax.axis_index("s") * NC + lax.axis_index("c")
    eb = wid * EPW

    pltpu.sync_copy(ids_h.at[pl.ds(wid * CPW, CPW)], idxc)
    hs = [
        pltpu.async_copy(tab_h.at[idxc.at[pl.ds(j * ICH, ICH)]],
                         rows.at[pl.ds(j * ICH, ICH)], sem)
        for j in range(NCH)
    ]
    for h in hs:
      h.wait()

    one16 = jnp.where(lax.iota(jnp.int32, 16) == 0, 1.0, 0.0)
    z16 = jnp.zeros((16,), jnp.float32)

    def esum(e, carry):
      for d in range(D // 16):
        sl = pl.ds(d * 16, 16)
        acc = rows[e * C, sl]
        for c2 in range(1, C):
          acc = acc + rows[e * C + c2, sl]
        xout[e, sl] = acc
      xout[e, pl.ds(D, 16)] = one16
      for d in range(D // 16 + 1, 128 // 16):
        xout[e, pl.ds(d * 16, 16)] = z16
      return carry

    lax.fori_loop(0, EPW, esum, 0)
    pltpu.sync_copy(xout, x_h.at[pl.ds(eb, EPW)])

  return k(ctx_ids, in_tab)


def _sc_label_gather(out_tab, bias128, lab_ids, brow_ids):
  mesh = plsc.VectorSubcoreMesh(core_axis_name="c", subcore_axis_name="s")

  @functools.partial(
      pl.kernel,
      out_type=(
          jax.ShapeDtypeStruct((B, 128), jnp.float32),
          jax.ShapeDtypeStruct((B, 128), jnp.float32),
      ),
      mesh=mesh,
      compiler_params=_SC_PARAMS,
      scratch_types=[
          pltpu.VMEM((EPW,), jnp.int32),
          pltpu.VMEM((EPW, D), jnp.float32),
          pltpu.VMEM((EPW,), jnp.int32),
          pltpu.VMEM((EPW, 128), jnp.float32),
          pltpu.VMEM((EPW, 128), jnp.float32),
          pltpu.SemaphoreType.DMA,
      ],
  )
  def k(tab_h, b128_h, lab_h, rid_h, le_h, br_h,
        idxl, labv, ridv, brv, mix, sem):
    wid = lax.axis_index("s") * NC + lax.axis_index("c")
    eb = wid * EPW
    pltpu.sync_copy(lab_h.at[pl.ds(eb, EPW)], idxl)
    pltpu.sync_copy(rid_h.at[pl.ds(eb, EPW)], ridv)
    h2 = pltpu.async_copy(tab_h.at[idxl], labv, sem)
    h3 = pltpu.async_copy(b128_h.at[ridv], brv, sem)
    h2.wait()
    h3.wait()

    z16 = jnp.zeros((16,), jnp.float32)

    def pack(e, carry):
      for d in range(D // 16):
        mix[e, pl.ds(d * 16, 16)] = labv[e, pl.ds(d * 16, 16)]
      for d in range(D // 16, 128 // 16):
        mix[e, pl.ds(d * 16, 16)] = z16
      return carry

    lax.fori_loop(0, EPW, pack, 0)
    pltpu.sync_copy(mix, le_h.at[pl.ds(eb, EPW)])
    pltpu.sync_copy(brv, br_h.at[pl.ds(eb, EPW)])

  return k(out_tab, bias128, lab_ids, brow_ids)


def _lse_body(x_ref, w_ref, be_ref, o_ref, xb_ref, m_ref, s_ref):
  i = pl.program_id(0)
  nv = pl.num_programs(0)

  @pl.when(i == 0)
  def _():
    xb_ref[...] = x_ref[:, 0:D].astype(jnp.bfloat16)
    m_ref[...] = jnp.full((B, 1), NEG, jnp.float32)
    s_ref[...] = jnp.zeros((B, 1), jnp.float32)

  xb = xb_ref[...]
  w = w_ref[...]

  def _mask(wv):
    row = i * TV + lax.broadcasted_iota(jnp.int32, (TV, 1), 0)
    return jnp.where(row < V, wv, jnp.bfloat16(0))

  w = lax.cond(i == nv - 1, _mask, lambda wv: wv, w)
  t = lax.dot_general(xb, w, (((1,), (1,)), ((), ())),
                      preferred_element_type=jnp.float32).astype(jnp.bfloat16)
  t = t + be_ref[...]
  tmax = jnp.max(t, axis=1, keepdims=True).astype(jnp.float32)
  m_old = m_ref[...]
  m_new = jnp.maximum(m_old, tmax)
  mb = m_new.astype(jnp.bfloat16)
  p = jnp.sum(jnp.exp(t - mb).astype(jnp.float32), axis=1, keepdims=True)
  s_ref[...] = s_ref[...] * jnp.exp(m_old - m_new) + p
  m_ref[...] = m_new

  @pl.when(i == nv - 1)
  def _():
    o_ref[...] = m_ref[...] + jnp.log(s_ref[...])


def _tc_lse(x_aug, w_bf, bias2):
  return pl.pallas_call(
      _lse_body,
      grid=(NVT,),
      in_specs=[
          pl.BlockSpec((B, 128), lambda i: (0, 0)),
          pl.BlockSpec((TV, D), lambda i: (i, 0)),
          pl.BlockSpec((1, TV), lambda i: (0, i)),
      ],
      out_specs=pl.BlockSpec((B, 1), lambda i: (0, 0)),
      out_shape=jax.ShapeDtypeStruct((B, 1), jnp.float32),
      scratch_shapes=[
          pltpu.VMEM((B, D), jnp.bfloat16),
          pltpu.VMEM((B, 1), jnp.float32),
          pltpu.VMEM((B, 1), jnp.float32),
      ],
  )(x_aug, w_bf, bias2)


def _final_body(lse_ref, x_ref, le_ref, br_ref, bl_ref, o_ref):
  xa = x_ref[:, 0:D]
  lab_logit = jnp.sum(xa * le_ref[:, 0:D], axis=1, keepdims=True)
  lsel = lax.broadcasted_iota(jnp.int32, (B, 128), 1) == bl_ref[...]
  lbias = jnp.sum(jnp.where(lsel, br_ref[...], 0.0), axis=1, keepdims=True)
  o_ref[...] = lse_ref[...] - lab_logit - lbias


def _tc_final(lse, x_aug, labv, brv, blane2):
  return pl.pallas_call(
      _final_body,
      out_shape=jax.ShapeDtypeStruct((B, 1), jnp.float32),
  )(lse, x_aug, labv, brv, blane2)


def kernel(input_word_ids, output_word_ids, input_layer_embeddings,
           output_layer_embeddings, output_layer_bias):
  ctx = input_word_ids.reshape(B * C)
  brow_ids = lax.shift_right_logical(output_word_ids, 7)
  blane = lax.bitwise_and(output_word_ids, 127)
  bias128 = jnp.pad(output_layer_bias, (0, VP - V)).reshape(BT, 128)
  bias2 = jnp.pad(output_layer_bias, (0, VP - V),
                  constant_values=NEG).reshape(1, VP).astype(jnp.bfloat16)

  w_bf = output_layer_embeddings.astype(jnp.bfloat16)

  x_aug = _sc_ctx_gather(ctx, input_layer_embeddings)
  labv, brv = _sc_label_gather(output_layer_embeddings, bias128,
                               output_word_ids, brow_ids)

  lse = _tc_lse(x_aug, w_bf, bias2)
  loss = _tc_final(lse, x_aug, labv, brv, blane.reshape(B, 1))
  return loss.reshape(B)


# speedup vs baseline: 1.0462x; 1.0018x over previous
import functools

import jax
import jax.numpy as jnp
from jax import lax
from jax.experimental import pallas as pl
from jax.experimental.pallas import tpu as pltpu
from jax.experimental.pallas import tpu_sc as plsc

V = 100000
D = 64
B = 1024
C = 20

NC = 2
NS = 16
NW = NC * NS
EPW = B // NW
CPW = EPW * C
ICH = 128
NCH = CPW // ICH

TV = 2048
NVT = (V + TV - 1) // TV
VP = NVT * TV
BT = VP // 128

_SC_PARAMS = pltpu.CompilerParams(use_tc_tiling_on_sc=False)
NEG = -1e30


def _sc_ctx_gather(ctx_ids, in_tab):
  mesh = plsc.VectorSubcoreMesh(core_axis_name="c", subcore_axis_name="s")

  @functools.partial(
      pl.kernel,
      out_type=jax.ShapeDtypeStruct((B, 128), jnp.float32),
      mesh=mesh,
      compiler_params=_SC_PARAMS,
      scratch_types=[
          pltpu.VMEM((CPW,), jnp.int32),
          pltpu.VMEM((CPW, D), jnp.float32),
          pltpu.VMEM((EPW, 128), jnp.float32),
          pltpu.SemaphoreType.DMA,
      ],
  )
  def k(ids_h, tab_h, x_h, idxc, rows, xout, sem):
    wid = lax.axis_index("s") * NC + lax.axis_index("c")
    eb = wid * EPW

    pltpu.sync_copy(ids_h.at[pl.ds(wid * CPW, CPW)], idxc)
    hs = [
        pltpu.async_copy(tab_h.at[idxc.at[pl.ds(j * ICH, ICH)]],
                         rows.at[pl.ds(j * ICH, ICH)], sem)
        for j in range(NCH)
    ]
    for h in hs:
      h.wait()

    one16 = jnp.where(lax.iota(jnp.int32, 16) == 0, 1.0, 0.0)
    z16 = jnp.zeros((16,), jnp.float32)

    def esum(e, carry):
      for d in range(D // 16):
        sl = pl.ds(d * 16, 16)
        acc = rows[e * C, sl]
        for c2 in range(1, C):
          acc = acc + rows[e * C + c2, sl]
        xout[e, sl] = acc
      xout[e, pl.ds(D, 16)] = one16
      for d in range(D // 16 + 1, 128 // 16):
        xout[e, pl.ds(d * 16, 16)] = z16
      return carry

    lax.fori_loop(0, EPW, esum, 0)
    pltpu.sync_copy(xout, x_h.at[pl.ds(eb, EPW)])

  return k(ctx_ids, in_tab)


def _sc_label_gather(out_tab, bias128, lab_ids, brow_ids, x_dep):
  mesh = plsc.VectorSubcoreMesh(core_axis_name="c", subcore_axis_name="s")

  @functools.partial(
      pl.kernel,
      out_type=(
          jax.ShapeDtypeStruct((B, 128), jnp.float32),
          jax.ShapeDtypeStruct((B, 128), jnp.float32),
      ),
      mesh=mesh,
      compiler_params=_SC_PARAMS,
      scratch_types=[
          pltpu.VMEM((EPW,), jnp.int32),
          pltpu.VMEM((EPW, D), jnp.float32),
          pltpu.VMEM((EPW,), jnp.int32),
          pltpu.VMEM((EPW, 128), jnp.float32),
          pltpu.VMEM((EPW, 128), jnp.float32),
          pltpu.SemaphoreType.DMA,
      ],
  )
  def k(tab_h, b128_h, lab_h, rid_h, xdep_h, le_h, br_h,
        idxl, labv, ridv, brv, mix, sem):
    wid = lax.axis_index("s") * NC + lax.axis_index("c")
    eb = wid * EPW
    pltpu.sync_copy(lab_h.at[pl.ds(eb, EPW)], idxl)
    pltpu.sync_copy(rid_h.at[pl.ds(eb, EPW)], ridv)
    h2 = pltpu.async_copy(tab_h.at[idxl], labv, sem)
    h3 = pltpu.async_copy(b128_h.at[ridv], brv, sem)
    h2.wait()
    h3.wait()

    z16 = jnp.zeros((16,), jnp.float32)

    def pack(e, carry):
      for d in range(D // 16):
        mix[e, pl.ds(d * 16, 16)] = labv[e, pl.ds(d * 16, 16)]
      for d in range(D // 16, 128 // 16):
        mix[e, pl.ds(d * 16, 16)] = z16
      return carry

    lax.fori_loop(0, EPW, pack, 0)
    pltpu.sync_copy(mix, le_h.at[pl.ds(eb, EPW)])
    pltpu.sync_copy(brv, br_h.at[pl.ds(eb, EPW)])

  return k(out_tab, bias128, lab_ids, brow_ids, x_dep)


def _lse_body(x_ref, w_ref, be_ref, o_ref, xb_ref, m_ref, s_ref):
  i = pl.program_id(0)
  nv = pl.num_programs(0)

  @pl.when(i == 0)
  def _():
    xb_ref[...] = x_ref[:, 0:D].astype(jnp.bfloat16)
    m_ref[...] = jnp.full((B, 1), NEG, jnp.float32)
    s_ref[...] = jnp.zeros((B, 1), jnp.float32)

  xb = xb_ref[...]
  w = w_ref[...]

  def _mask(wv):
    row = i * TV + lax.broadcasted_iota(jnp.int32, (TV, 1), 0)
    return jnp.where(row < V, wv, jnp.bfloat16(0))

  w = lax.cond(i == nv - 1, _mask, lambda wv: wv, w)
  t = lax.dot_general(xb, w, (((1,), (1,)), ((), ())),
                      preferred_element_type=jnp.float32).astype(jnp.bfloat16)
  t = t + be_ref[...]
  tmax = jnp.max(t, axis=1, keepdims=True).astype(jnp.float32)
  m_old = m_ref[...]
  m_new = jnp.maximum(m_old, tmax)
  mb = m_new.astype(jnp.bfloat16)
  p = jnp.sum(jnp.exp(t - mb).astype(jnp.float32), axis=1, keepdims=True)
  s_ref[...] = s_ref[...] * jnp.exp(m_old - m_new) + p
  m_ref[...] = m_new

  @pl.when(i == nv - 1)
  def _():
    o_ref[...] = m_ref[...] + jnp.log(s_ref[...])


def _tc_lse(x_aug, w_bf, bias2):
  return pl.pallas_call(
      _lse_body,
      grid=(NVT,),
      in_specs=[
          pl.BlockSpec((B, 128), lambda i: (0, 0)),
          pl.BlockSpec((TV, D), lambda i: (i, 0)),
          pl.BlockSpec((1, TV), lambda i: (0, i)),
      ],
      out_specs=pl.BlockSpec((B, 1), lambda i: (0, 0)),
      out_shape=jax.ShapeDtypeStruct((B, 1), jnp.float32),
      scratch_shapes=[
          pltpu.VMEM((B, D), jnp.bfloat16),
          pltpu.VMEM((B, 1), jnp.float32),
          pltpu.VMEM((B, 1), jnp.float32),
      ],
  )(x_aug, w_bf, bias2)


def _final_body(lse_ref, x_ref, le_ref, br_ref, bl_ref, o_ref):
  xa = x_ref[:, 0:D]
  lab_logit = jnp.sum(xa * le_ref[:, 0:D], axis=1, keepdims=True)
  lsel = lax.broadcasted_iota(jnp.int32, (B, 128), 1) == bl_ref[...]
  lbias = jnp.sum(jnp.where(lsel, br_ref[...], 0.0), axis=1, keepdims=True)
  o_ref[...] = lse_ref[...] - lab_logit - lbias


def _tc_final(lse, x_aug, labv, brv, blane2):
  return pl.pallas_call(
      _final_body,
      out_shape=jax.ShapeDtypeStruct((B, 1), jnp.float32),
  )(lse, x_aug, labv, brv, blane2)


def kernel(input_word_ids, output_word_ids, input_layer_embeddings,
           output_layer_embeddings, output_layer_bias):
  ctx = input_word_ids.reshape(B * C)
  brow_ids = lax.shift_right_logical(output_word_ids, 7)
  blane = lax.bitwise_and(output_word_ids, 127)
  bias128 = jnp.pad(output_layer_bias, (0, VP - V)).reshape(BT, 128)
  bias2 = jnp.pad(output_layer_bias, (0, VP - V),
                  constant_values=NEG).reshape(1, VP).astype(jnp.bfloat16)

  w_bf = output_layer_embeddings.astype(jnp.bfloat16)

  x_aug = _sc_ctx_gather(ctx, input_layer_embeddings)
  labv, brv = _sc_label_gather(output_layer_embeddings, bias128,
                               output_word_ids, brow_ids, x_aug)

  lse = _tc_lse(x_aug, w_bf, bias2)
  loss = _tc_final(lse, x_aug, labv, brv, blane.reshape(B, 1))
  return loss.reshape(B)


# speedup vs baseline: 1.0815x; 1.0337x over previous
import functools

import jax
import jax.numpy as jnp
from jax import lax
from jax.experimental import pallas as pl
from jax.experimental.pallas import tpu as pltpu
from jax.experimental.pallas import tpu_sc as plsc

V = 100000
D = 64
B = 1024
C = 20

NC = 2
NS = 16
NW = NC * NS
EPW = B // NW
CPW = EPW * C
ICH = 128
NCH = CPW // ICH

VH = V // 2
TP = 1024
NPT = (VH + TP - 1) // TP
HP = NPT * TP
BT = 2 * HP // 128

_SC_PARAMS = pltpu.CompilerParams(use_tc_tiling_on_sc=True,
                                  needs_layout_passes=False)
NEG = -1e30


def _prepw_body(w_ref, o_ref):
  i = pl.program_id(0)
  nv = pl.num_programs(0)
  w = w_ref[...]

  def _mask(wv):
    row = i * TP + lax.broadcasted_iota(jnp.int32, (TP, 1), 0)
    return jnp.where(row < VH, wv, 0.0)

  w = lax.cond(i == nv - 1, _mask, lambda wv: wv, w)
  o_ref[...] = w.astype(jnp.bfloat16)


def _prep_w(out_pair):
  return pl.pallas_call(
      _prepw_body,
      grid=(NPT,),
      in_specs=[pl.BlockSpec((TP, 128), lambda i: (i, 0))],
      out_specs=pl.BlockSpec((TP, 128), lambda i: (i, 0)),
      out_shape=jax.ShapeDtypeStruct((HP, 128), jnp.bfloat16),
  )(out_pair)


def _sc_ctx_gather(ctx_prow, ctx_par, in_pair):
  mesh = plsc.VectorSubcoreMesh(core_axis_name="c", subcore_axis_name="s")

  @functools.partial(
      pl.kernel,
      out_type=jax.ShapeDtypeStruct((B, 128), jnp.float32),
      mesh=mesh,
      compiler_params=_SC_PARAMS,
      scratch_types=[
          pltpu.VMEM((CPW,), jnp.int32),
          pltpu.VMEM((CPW,), jnp.int32),
          pltpu.VMEM((CPW, 128), jnp.float32),
          pltpu.VMEM((EPW, 128), jnp.float32),
          pltpu.SemaphoreType.DMA,
      ],
  )
  def k(prow_h, par_h, pair_h, x_h, idxc, parv, rows, xout, sem):
    wid = lax.axis_index("s") * NC + lax.axis_index("c")
    eb = wid * EPW

    pltpu.sync_copy(prow_h.at[pl.ds(wid * CPW, CPW)], idxc)
    pltpu.sync_copy(par_h.at[pl.ds(wid * CPW, CPW)], parv)
    hs = [
        pltpu.async_copy(pair_h.at[idxc.at[pl.ds(j * ICH, ICH)]],
                         rows.at[pl.ds(j * ICH, ICH)], sem)
        for j in range(NCH)
    ]
    for h in hs:
      h.wait()

    one16 = jnp.where(lax.iota(jnp.int32, 16) == 0, 1.0, 0.0)
    z16 = jnp.zeros((16,), jnp.float32)

    def esum(e, carry):
      masks = []
      for c2 in range(C):
        psplat = plsc.load_gather(parv, [jnp.full((16,), e * C + c2,
                                                  jnp.int32)])
        masks.append(psplat == 1)
      accs = []
      for d in range(D // 16):
        r = e * C
        acc = jnp.where(masks[0], rows[r, pl.ds(D + d * 16, 16)],
                        rows[r, pl.ds(d * 16, 16)])
        for c2 in range(1, C):
          r = e * C + c2
          acc = acc + jnp.where(masks[c2], rows[r, pl.ds(D + d * 16, 16)],
                                rows[r, pl.ds(d * 16, 16)])
        accs.append(acc)
      for d in range(D // 16):
        xout[e, pl.ds(d * 16, 16)] = accs[d]
      xout[e, pl.ds(D, 16)] = one16
      for d in range(D // 16 + 1, 128 // 16):
        xout[e, pl.ds(d * 16, 16)] = z16
      return carry

    lax.fori_loop(0, EPW, esum, 0)
    pltpu.sync_copy(xout, x_h.at[pl.ds(eb, EPW)])

  return k(ctx_prow, ctx_par, in_pair)


def _sc_label_gather(out_pair, bias128, lab_prow, brow_ids, x_dep):
  mesh = plsc.VectorSubcoreMesh(core_axis_name="c", subcore_axis_name="s")

  @functools.partial(
      pl.kernel,
      out_type=(
          jax.ShapeDtypeStruct((B, 128), jnp.float32),
          jax.ShapeDtypeStruct((B, 128), jnp.float32),
      ),
      mesh=mesh,
      compiler_params=_SC_PARAMS,
      scratch_types=[
          pltpu.VMEM((EPW,), jnp.int32),
          pltpu.VMEM((EPW, 128), jnp.float32),
          pltpu.VMEM((EPW,), jnp.int32),
          pltpu.VMEM((EPW, 128), jnp.float32),
          pltpu.SemaphoreType.DMA,
      ],
  )
  def k(pair_h, b128_h, lab_h, rid_h, xdep_h,
        le_h, br_h, idxl, labv, ridv, brv, sem):
    wid = lax.axis_index("s") * NC + lax.axis_index("c")
    eb = wid * EPW
    pltpu.sync_copy(lab_h.at[pl.ds(eb, EPW)], idxl)
    pltpu.sync_copy(rid_h.at[pl.ds(eb, EPW)], ridv)
    h2 = pltpu.async_copy(pair_h.at[idxl], labv, sem)
    h3 = pltpu.async_copy(b128_h.at[ridv], brv, sem)
    h2.wait()
    h3.wait()
    pltpu.sync_copy(labv, le_h.at[pl.ds(eb, EPW)])
    pltpu.sync_copy(brv, br_h.at[pl.ds(eb, EPW)])

  return k(out_pair, bias128, lab_prow, brow_ids, x_dep)


def _lse_body(x_ref, w_ref, be_ref, bo_ref, o_ref, xb_ref, m_ref, s_ref):
  i = pl.program_id(0)
  nv = pl.num_programs(0)

  @pl.when(i == 0)
  def _():
    xb_ref[...] = x_ref[:, 0:D].astype(jnp.bfloat16)
    m_ref[...] = jnp.full((B, 1), NEG, jnp.float32)
    s_ref[...] = jnp.zeros((B, 1), jnp.float32)

  xb = xb_ref[...]
  te = lax.dot_general(xb, w_ref[:, 0:D], (((1,), (1,)), ((), ())),
                       preferred_element_type=jnp.float32).astype(jnp.bfloat16)
  to = lax.dot_general(xb, w_ref[:, D:2 * D], (((1,), (1,)), ((), ())),
                       preferred_element_type=jnp.float32).astype(jnp.bfloat16)
  te = te + be_ref[...]
  to = to + bo_ref[...]
  tmax = jnp.maximum(
      jnp.max(te, axis=1, keepdims=True),
      jnp.max(to, axis=1, keepdims=True)).astype(jnp.float32)
  m_old = m_ref[...]
  m_new = jnp.maximum(m_old, tmax)
  mb = m_new.astype(jnp.bfloat16)
  p = (jnp.sum(jnp.exp(te - mb).astype(jnp.float32), axis=1, keepdims=True) +
       jnp.sum(jnp.exp(to - mb).astype(jnp.float32), axis=1, keepdims=True))
  s_ref[...] = s_ref[...] * jnp.exp(m_old - m_new) + p
  m_ref[...] = m_new

  @pl.when(i == nv - 1)
  def _():
    o_ref[...] = m_ref[...] + jnp.log(s_ref[...])


def _tc_lse(x_aug, w_pair, be, bo):
  return pl.pallas_call(
      _lse_body,
      grid=(NPT,),
      in_specs=[
          pl.BlockSpec((B, 128), lambda i: (0, 0)),
          pl.BlockSpec((TP, 128), lambda i: (i, 0)),
          pl.BlockSpec((1, TP), lambda i: (0, i)),
          pl.BlockSpec((1, TP), lambda i: (0, i)),
      ],
      out_specs=pl.BlockSpec((B, 1), lambda i: (0, 0)),
      out_shape=jax.ShapeDtypeStruct((B, 1), jnp.float32),
      scratch_shapes=[
          pltpu.VMEM((B, D), jnp.bfloat16),
          pltpu.VMEM((B, 1), jnp.float32),
          pltpu.VMEM((B, 1), jnp.float32),
      ],
  )(x_aug, w_pair, be, bo)


def _final_body(lse_ref, x_ref, le_ref, br_ref, ph_ref, bl_ref, o_ref):
  xa = x_ref[:, 0:D]
  sel_hi = ph_ref[...] == 1
  lv = jnp.where(sel_hi, le_ref[:, D:2 * D], le_ref[:, 0:D])
  lab_logit = jnp.sum(xa * lv, axis=1, keepdims=True)
  lsel = lax.broadcasted_iota(jnp.int32, (B, 128), 1) == bl_ref[...]
  lbias = jnp.sum(jnp.where(lsel, br_ref[...], 0.0), axis=1, keepdims=True)
  o_ref[...] = lse_ref[...] - lab_logit - lbias


def _tc_final(lse, x_aug, labv, brv, phalf2, blane2):
  return pl.pallas_call(
      _final_body,
      out_shape=jax.ShapeDtypeStruct((B, 1), jnp.float32),
  )(lse, x_aug, labv, brv, phalf2, blane2)


def kernel(input_word_ids, output_word_ids, input_layer_embeddings,
           output_layer_embeddings, output_layer_bias):
  ctx = input_word_ids.reshape(B * C)
  ctx_prow = lax.shift_right_logical(ctx, 1)
  ctx_par = lax.bitwise_and(ctx, 1)
  lab_prow = lax.shift_right_logical(output_word_ids, 1)
  lab_phalf = lax.bitwise_and(output_word_ids, 1)
  brow_ids = lax.shift_right_logical(output_word_ids, 7)
  blane = lax.bitwise_and(output_word_ids, 127)

  in_pair = input_layer_embeddings.reshape(VH, 128)
  out_pair = output_layer_embeddings.reshape(VH, 128)
  bias128 = jnp.pad(output_layer_bias, (0, 2 * HP - V)).reshape(BT, 128)
  be = jnp.pad(output_layer_bias[0::2], (0, HP - VH),
               constant_values=NEG).reshape(1, HP).astype(jnp.bfloat16)
  bo = jnp.pad(output_layer_bias[1::2], (0, HP - VH),
               constant_values=NEG).reshape(1, HP).astype(jnp.bfloat16)

  w_pair = _prep_w(out_pair)
  x_aug = _sc_ctx_gather(ctx_prow, ctx_par, in_pair)
  labv, brv = _sc_label_gather(out_pair, bias128, lab_prow, brow_ids, x_aug)

  lse = _tc_lse(x_aug, w_pair, be, bo)
  loss = _tc_final(lse, x_aug, labv, brv,
                   lab_phalf.reshape(B, 1), blane.reshape(B, 1))
  return loss.reshape(B)
